# baseline (speedup 1.0000x reference)
import functools

import jax
import jax.numpy as jnp
from jax import lax
from jax.experimental import pallas as pl
from jax.experimental.pallas import tpu as pltpu
from jax.experimental.pallas import tpu_sc as plsc

B = 4096
L = 50
V = 1000000
D = 64
NC_OUT = 3

_NUM_CORES = 2
_NUM_SUBCORES = 16
_NW = _NUM_CORES * _NUM_SUBCORES
_TASKS = 2 * B
_TPW = _TASKS // _NW
_G = 4
_NG = _TPW // _G
_ROWS_G = _G * L
_PAIRS_G = _G // 2
_LANES = 16
_DCH = D // _LANES
_RBLK = 512
_NST = 977
_TH = _NST * _RBLK

_GDN = lax.GatherDimensionNumbers(
    offset_dims=(), collapsed_slice_dims=(0,), start_index_map=(0,))
_IN_BOUNDS = lax.GatherScatterMode.PROMISE_IN_BOUNDS


def _repack(table_t):
    def body(a_ref, b_ref, o_ref):
        eye = (lax.broadcasted_iota(jnp.int32, (D, D), 0)
               == lax.broadcasted_iota(jnp.int32, (D, D), 1)
               ).astype(jnp.float32)
        dn = (((0,), (0,)), ((), ()))
        o_ref[:, 0:D] = lax.dot_general(
            a_ref[...], eye, dn, preferred_element_type=jnp.float32)
        o_ref[:, D:2 * D] = lax.dot_general(
            b_ref[...], eye, dn, preferred_element_type=jnp.float32)

    return pl.pallas_call(
        body,
        grid=(_NST,),
        in_specs=[
            pl.BlockSpec((D, _RBLK), lambda i: (0, i)),
            pl.BlockSpec((D, _RBLK), lambda i: (0, i + _NST)),
        ],
        out_specs=pl.BlockSpec((_RBLK, 2 * D), lambda i: (i, 0)),
        out_shape=jax.ShapeDtypeStruct((_TH, 2 * D), jnp.float32),
    )(table_t, table_t)


def _make_sc_pool():
    mesh = plsc.VectorSubcoreMesh(core_axis_name="c", subcore_axis_name="s")

    @functools.partial(
        pl.kernel,
        out_type=jax.ShapeDtypeStruct((B, 2 * D), jnp.float32),
        mesh=mesh,
        compiler_params=pltpu.CompilerParams(
            needs_layout_passes=False, use_tc_tiling_on_sc=True),
        scratch_types=[
            pltpu.VMEM((_TPW * L,), jnp.int32),
            pltpu.VMEM((_TPW * L + 16,), jnp.float32),
            pltpu.VMEM((_TPW * L + 16,), jnp.float32),
            pltpu.VMEM((_ROWS_G, 2 * D), jnp.float32),
            pltpu.VMEM((_ROWS_G, 2 * D), jnp.float32),
            pltpu.VMEM((_PAIRS_G, 2 * D), jnp.float32),
            pltpu.SemaphoreType.DMA,
            pltpu.SemaphoreType.DMA,
        ],
    )
    def pool(s_hbm, wl_hbm, wh_hbm, table_hbm, out_hbm,
             idx_v, wlo_v, whi_v, rows_a, rows_b, outb_v, sem_a, sem_b):
        wid = lax.axis_index("s") * _NUM_CORES + lax.axis_index("c")
        ebase = wid * (_TPW * L)

        pltpu.sync_copy(s_hbm.at[pl.ds(ebase, _TPW * L)], idx_v)
        pltpu.sync_copy(wl_hbm.at[pl.ds(ebase, _TPW * L)],
                        wlo_v.at[pl.ds(0, _TPW * L)])
        pltpu.sync_copy(wh_hbm.at[pl.ds(ebase, _TPW * L)],
                        whi_v.at[pl.ds(0, _TPW * L)])

        def issue(g, rows, sem):
            base = g * _ROWS_G
            for off, n in ((0, 128), (128, _ROWS_G - 128)):
                pltpu.async_copy(
                    table_hbm.at[idx_v.at[pl.ds(base + off, n)]],
                    rows.at[pl.ds(off, n)],
                    sem,
                )

        def wait(rows, sem):
            pltpu.make_async_copy(table_hbm.at[pl.ds(0, _ROWS_G)], rows, sem).wait()

        def wbcast(chunk, j):
            return lax.gather(chunk, jnp.full((_LANES, 1), j, jnp.int32),
                              _GDN, (1,), mode=_IN_BOUNDS)

        def pooled(rows, woff, rbase):
            def fma_rows(accs, wl, wh, r0, nrows):
                lo, hi = accs
                for j in range(nrows):
                    wvl = wbcast(wl, j)
                    wvh = wbcast(wh, j)
                    r = r0 + j
                    lo = tuple(
                        lo[c] + wvl * rows[r, pl.ds(c * _LANES, _LANES)]
                        for c in range(_DCH))
                    hi = tuple(
                        hi[c] + wvh * rows[r, pl.ds(D + c * _LANES, _LANES)]
                        for c in range(_DCH))
                return (lo, hi)

            def body(k, accs):
                wl = wlo_v[pl.ds(woff + k * _LANES, _LANES)]
                wh = whi_v[pl.ds(woff + k * _LANES, _LANES)]
                return fma_rows(accs, wl, wh, rbase + k * _LANES, _LANES)

            z = jnp.zeros((_LANES,), jnp.float32)
            accs = lax.fori_loop(0, L // _LANES, body,
                                 ((z,) * _DCH, (z,) * _DCH))
            ntail = L - _LANES * (L // _LANES)
            wl = wlo_v[pl.ds(woff + L - ntail, _LANES)]
            wh = whi_v[pl.ds(woff + L - ntail, _LANES)]
            lo, hi = fma_rows(accs, wl, wh, rbase + L - ntail, ntail)
            return tuple(lo[c] + hi[c] for c in range(_DCH))

        def compute(g, rows):
            for q in range(_PAIRS_G):
                woff = (g * _G + 2 * q) * L
                e1 = pooled(rows, woff, (2 * q) * L)
                e2 = pooled(rows, woff + L, (2 * q + 1) * L)
                for c in range(_DCH):
                    outb_v[q, pl.ds(c * _LANES, _LANES)] = (
                        jnp.abs(e1[c] - e2[c]) * (1.0 / L))
                    outb_v[q, pl.ds(D + c * _LANES, _LANES)] = (
                        (e1[c] * e2[c]) * (1.0 / (L * L)))
            pair0 = wid * (_TPW // 2) + g * _PAIRS_G
            pltpu.sync_copy(outb_v, out_hbm.at[pl.ds(pair0, _PAIRS_G)])

        issue(0, rows_a, sem_a)
        issue(1, rows_b, sem_b)

        def step(i, carry):
            g0 = 2 * i
            wait(rows_a, sem_a)
            compute(g0, rows_a)

            @pl.when(i < _NG // 2 - 1)
            def _():
                issue(g0 + 2, rows_a, sem_a)

            wait(rows_b, sem_b)
            compute(g0 + 1, rows_b)

            @pl.when(i < _NG // 2 - 1)
            def _():
                issue(g0 + 3, rows_b, sem_b)

            return carry

        lax.fori_loop(0, _NG // 2, step, 0)

    return pool


def _mlp(x, W1, b1, W2, b2, W3, b3):
    def body(x_ref, w1_ref, b1_ref, w2_ref, b2_ref, w3_ref, b3_ref, o_ref):
        f32 = jnp.float32
        w12 = jnp.dot(w1_ref[...], w2_ref[...], preferred_element_type=f32)
        w123 = jnp.dot(w12, w3_ref[...], preferred_element_type=f32)
        b12 = jnp.dot(b1_ref[...], w2_ref[...], preferred_element_type=f32) + b2_ref[...]
        beff = jnp.dot(b12, w3_ref[...], preferred_element_type=f32) + b3_ref[...]
        o_ref[...] = jnp.dot(x_ref[...], w123, preferred_element_type=f32) + beff

    return pl.pallas_call(
        body,
        out_shape=jax.ShapeDtypeStruct((B, NC_OUT), jnp.float32),
    )(x, W1, b1.reshape(1, -1), W2, b2.reshape(1, -1), W3, b3.reshape(1, -1))


def kernel(s1, s2, w1, w2, table, W1, b1, W2, b2, W3, b3):
    s_all = jnp.stack([s1.astype(jnp.int32), s2.astype(jnp.int32)],
                      axis=1).reshape(-1)
    w_all = jnp.stack([w1, w2], axis=1).reshape(-1)
    hi = s_all >= _TH
    s_half = jnp.where(hi, s_all - _TH, s_all)
    w_lo = jnp.where(hi, 0.0, w_all)
    w_hi = jnp.where(hi, w_all, 0.0)
    table2 = _repack(table.T)
    combine = _make_sc_pool()(s_half, w_lo, w_hi, table2)
    return _mlp(combine, W1, b1, W2, b2, W3, b3)


# speedup vs baseline: 2.0396x; 2.0396x over previous
import functools

import jax
import jax.numpy as jnp
from jax import lax
from jax.experimental import pallas as pl
from jax.experimental.pallas import tpu as pltpu
from jax.experimental.pallas import tpu_sc as plsc

B = 4096
L = 50
V = 1000000
D = 64
NC_OUT = 3

_NUM_CORES = 2
_NUM_SUBCORES = 16
_NW = _NUM_CORES * _NUM_SUBCORES
_TASKS = 2 * B
_TPW = _TASKS // _NW
_G = 4
_NG = _TPW // _G
_ROWS_G = _G * L
_PAIRS_G = _G // 2
_LANES = 16
_DCH = D // _LANES
_RBLK = 8832
_NST = 57
_TH = _NST * _RBLK

_GDN = lax.GatherDimensionNumbers(
    offset_dims=(), collapsed_slice_dims=(0,), start_index_map=(0,))
_IN_BOUNDS = lax.GatherScatterMode.PROMISE_IN_BOUNDS


def _repack(table_t):
    def body(a_ref, b_ref, o_ref):
        eye = (lax.broadcasted_iota(jnp.int32, (D, D), 0)
               == lax.broadcasted_iota(jnp.int32, (D, D), 1)
               ).astype(jnp.float32)
        dn = (((0,), (0,)), ((), ()))
        o_ref[:, 0:D] = lax.dot_general(
            a_ref[...], eye, dn, preferred_element_type=jnp.float32)
        o_ref[:, D:2 * D] = lax.dot_general(
            b_ref[...], eye, dn, preferred_element_type=jnp.float32)

    return pl.pallas_call(
        body,
        grid=(_NST,),
        in_specs=[
            pl.BlockSpec((D, _RBLK), lambda i: (0, i)),
            pl.BlockSpec((D, _RBLK), lambda i: (0, i + _NST)),
        ],
        out_specs=pl.BlockSpec((_RBLK, 2 * D), lambda i: (i, 0)),
        out_shape=jax.ShapeDtypeStruct((_TH, 2 * D), jnp.float32),
    )(table_t, table_t)


def _make_sc_pool():
    mesh = plsc.VectorSubcoreMesh(core_axis_name="c", subcore_axis_name="s")

    @functools.partial(
        pl.kernel,
        out_type=jax.ShapeDtypeStruct((B, 2 * D), jnp.float32),
        mesh=mesh,
        compiler_params=pltpu.CompilerParams(
            needs_layout_passes=False, use_tc_tiling_on_sc=True),
        scratch_types=[
            pltpu.VMEM((_TPW * L,), jnp.int32),
            pltpu.VMEM((_TPW * L + 16,), jnp.float32),
            pltpu.VMEM((_TPW * L + 16,), jnp.float32),
            pltpu.VMEM((_ROWS_G, 2 * D), jnp.float32),
            pltpu.VMEM((_ROWS_G, 2 * D), jnp.float32),
            pltpu.VMEM((_PAIRS_G, 2 * D), jnp.float32),
            pltpu.SemaphoreType.DMA,
            pltpu.SemaphoreType.DMA,
        ],
    )
    def pool(s_hbm, wl_hbm, wh_hbm, table_hbm, out_hbm,
             idx_v, wlo_v, whi_v, rows_a, rows_b, outb_v, sem_a, sem_b):
        wid = lax.axis_index("s") * _NUM_CORES + lax.axis_index("c")
        ebase = wid * (_TPW * L)

        pltpu.sync_copy(s_hbm.at[pl.ds(ebase, _TPW * L)], idx_v)
        pltpu.sync_copy(wl_hbm.at[pl.ds(ebase, _TPW * L)],
                        wlo_v.at[pl.ds(0, _TPW * L)])
        pltpu.sync_copy(wh_hbm.at[pl.ds(ebase, _TPW * L)],
                        whi_v.at[pl.ds(0, _TPW * L)])

        def issue(g, rows, sem):
            base = g * _ROWS_G
            for off, n in ((0, 128), (128, _ROWS_G - 128)):
                pltpu.async_copy(
                    table_hbm.at[idx_v.at[pl.ds(base + off, n)]],
                    rows.at[pl.ds(off, n)],
                    sem,
                )

        def wait(rows, sem):
            pltpu.make_async_copy(table_hbm.at[pl.ds(0, _ROWS_G)], rows, sem).wait()

        def wbcast(chunk, j):
            return lax.gather(chunk, jnp.full((_LANES, 1), j, jnp.int32),
                              _GDN, (1,), mode=_IN_BOUNDS)

        def pooled(rows, woff, rbase):
            def fma_rows(accs, wl, wh, r0, nrows):
                lo, hi = accs
                for j in range(nrows):
                    wvl = wbcast(wl, j)
                    wvh = wbcast(wh, j)
                    r = r0 + j
                    lo = tuple(
                        lo[c] + wvl * rows[r, pl.ds(c * _LANES, _LANES)]
                        for c in range(_DCH))
                    hi = tuple(
                        hi[c] + wvh * rows[r, pl.ds(D + c * _LANES, _LANES)]
                        for c in range(_DCH))
                return (lo, hi)

            def body(k, accs):
                wl = wlo_v[pl.ds(woff + k * _LANES, _LANES)]
                wh = whi_v[pl.ds(woff + k * _LANES, _LANES)]
                return fma_rows(accs, wl, wh, rbase + k * _LANES, _LANES)

            z = jnp.zeros((_LANES,), jnp.float32)
            accs = lax.fori_loop(0, L // _LANES, body,
                                 ((z,) * _DCH, (z,) * _DCH))
            ntail = L - _LANES * (L // _LANES)
            wl = wlo_v[pl.ds(woff + L - ntail, _LANES)]
            wh = whi_v[pl.ds(woff + L - ntail, _LANES)]
            lo, hi = fma_rows(accs, wl, wh, rbase + L - ntail, ntail)
            return tuple(lo[c] + hi[c] for c in range(_DCH))

        def compute(g, rows):
            for q in range(_PAIRS_G):
                woff = (g * _G + 2 * q) * L
                e1 = pooled(rows, woff, (2 * q) * L)
                e2 = pooled(rows, woff + L, (2 * q + 1) * L)
                for c in range(_DCH):
                    outb_v[q, pl.ds(c * _LANES, _LANES)] = (
                        jnp.abs(e1[c] - e2[c]) * (1.0 / L))
                    outb_v[q, pl.ds(D + c * _LANES, _LANES)] = (
                        (e1[c] * e2[c]) * (1.0 / (L * L)))
            pair0 = wid * (_TPW // 2) + g * _PAIRS_G
            pltpu.sync_copy(outb_v, out_hbm.at[pl.ds(pair0, _PAIRS_G)])

        issue(0, rows_a, sem_a)
        issue(1, rows_b, sem_b)

        def step(i, carry):
            g0 = 2 * i
            wait(rows_a, sem_a)
            compute(g0, rows_a)

            @pl.when(i < _NG // 2 - 1)
            def _():
                issue(g0 + 2, rows_a, sem_a)

            wait(rows_b, sem_b)
            compute(g0 + 1, rows_b)

            @pl.when(i < _NG // 2 - 1)
            def _():
                issue(g0 + 3, rows_b, sem_b)

            return carry

        lax.fori_loop(0, _NG // 2, step, 0)

    return pool


def _mlp(x, W1, b1, W2, b2, W3, b3):
    def body(x_ref, w1_ref, b1_ref, w2_ref, b2_ref, w3_ref, b3_ref, o_ref):
        f32 = jnp.float32
        w12 = jnp.dot(w1_ref[...], w2_ref[...], preferred_element_type=f32)
        w123 = jnp.dot(w12, w3_ref[...], preferred_element_type=f32)
        b12 = jnp.dot(b1_ref[...], w2_ref[...], preferred_element_type=f32) + b2_ref[...]
        beff = jnp.dot(b12, w3_ref[...], preferred_element_type=f32) + b3_ref[...]
        o_ref[...] = jnp.dot(x_ref[...], w123, preferred_element_type=f32) + beff

    return pl.pallas_call(
        body,
        out_shape=jax.ShapeDtypeStruct((B, NC_OUT), jnp.float32),
    )(x, W1, b1.reshape(1, -1), W2, b2.reshape(1, -1), W3, b3.reshape(1, -1))


def kernel(s1, s2, w1, w2, table, W1, b1, W2, b2, W3, b3):
    s_all = jnp.stack([s1.astype(jnp.int32), s2.astype(jnp.int32)],
                      axis=1).reshape(-1)
    w_all = jnp.stack([w1, w2], axis=1).reshape(-1)
    hi = s_all >= _TH
    s_half = jnp.where(hi, s_all - _TH, s_all)
    w_lo = jnp.where(hi, 0.0, w_all)
    w_hi = jnp.where(hi, w_all, 0.0)
    table2 = _repack(table.T)
    combine = _make_sc_pool()(s_half, w_lo, w_hi, table2)
    return _mlp(combine, W1, b1, W2, b2, W3, b3)


# speedup vs baseline: 2.0982x; 1.0287x over previous
import functools

import jax
import jax.numpy as jnp
from jax import lax
from jax.experimental import pallas as pl
from jax.experimental.pallas import tpu as pltpu
from jax.experimental.pallas import tpu_sc as plsc

B = 4096
L = 50
V = 1000000
D = 64
NC_OUT = 3

_NUM_CORES = 2
_NUM_SUBCORES = 16
_NW = _NUM_CORES * _NUM_SUBCORES
_TASKS = 2 * B
_TPW = _TASKS // _NW
_G = 4
_NG = _TPW // _G
_ROWS_G = _G * L
_PAIRS_G = _G // 2
_LANES = 16
_DCH = D // _LANES
_RBLK = 17920
_NST = 28
_TH = _NST * _RBLK

_GDN = lax.GatherDimensionNumbers(
    offset_dims=(), collapsed_slice_dims=(0,), start_index_map=(0,))
_IN_BOUNDS = lax.GatherScatterMode.PROMISE_IN_BOUNDS


def _repack(table_t):
    def body(a_ref, b_ref, o_ref):
        eye = (lax.broadcasted_iota(jnp.int32, (D, D), 0)
               == lax.broadcasted_iota(jnp.int32, (D, D), 1)
               ).astype(jnp.float32)
        dn = (((0,), (0,)), ((), ()))
        o_ref[:, 0:D] = lax.dot_general(
            a_ref[...], eye, dn, preferred_element_type=jnp.float32)
        o_ref[:, D:2 * D] = lax.dot_general(
            b_ref[...], eye, dn, preferred_element_type=jnp.float32)

    return pl.pallas_call(
        body,
        grid=(_NST,),
        in_specs=[
            pl.BlockSpec((D, _RBLK), lambda i: (0, i)),
            pl.BlockSpec((D, _RBLK), lambda i: (0, i + _NST)),
        ],
        out_specs=pl.BlockSpec((_RBLK, 2 * D), lambda i: (i, 0)),
        out_shape=jax.ShapeDtypeStruct((_TH, 2 * D), jnp.float32),
    )(table_t, table_t)


def _make_sc_pool():
    mesh = plsc.VectorSubcoreMesh(core_axis_name="c", subcore_axis_name="s")

    @functools.partial(
        pl.kernel,
        out_type=jax.ShapeDtypeStruct((B, 2 * D), jnp.float32),
        mesh=mesh,
        compiler_params=pltpu.CompilerParams(
            needs_layout_passes=False, use_tc_tiling_on_sc=True),
        scratch_types=[
            pltpu.VMEM((_TPW * L,), jnp.int32),
            pltpu.VMEM((_TPW * L + 16,), jnp.float32),
            pltpu.VMEM((_TPW * L + 16,), jnp.float32),
            pltpu.VMEM((_ROWS_G, 2 * D), jnp.float32),
            pltpu.VMEM((_ROWS_G, 2 * D), jnp.float32),
            pltpu.VMEM((_PAIRS_G, 2 * D), jnp.float32),
            pltpu.SemaphoreType.DMA,
            pltpu.SemaphoreType.DMA,
        ],
    )
    def pool(s_hbm, wl_hbm, wh_hbm, table_hbm, out_hbm,
             idx_v, wlo_v, whi_v, rows_a, rows_b, outb_v, sem_a, sem_b):
        wid = lax.axis_index("s") * _NUM_CORES + lax.axis_index("c")
        ebase = wid * (_TPW * L)

        pltpu.sync_copy(s_hbm.at[pl.ds(ebase, _TPW * L)], idx_v)
        pltpu.sync_copy(wl_hbm.at[pl.ds(ebase, _TPW * L)],
                        wlo_v.at[pl.ds(0, _TPW * L)])
        pltpu.sync_copy(wh_hbm.at[pl.ds(ebase, _TPW * L)],
                        whi_v.at[pl.ds(0, _TPW * L)])

        def issue(g, rows, sem):
            base = g * _ROWS_G
            for off, n in ((0, 128), (128, _ROWS_G - 128)):
                pltpu.async_copy(
                    table_hbm.at[idx_v.at[pl.ds(base + off, n)]],
                    rows.at[pl.ds(off, n)],
                    sem,
                )

        def wait(rows, sem):
            pltpu.make_async_copy(table_hbm.at[pl.ds(0, _ROWS_G)], rows, sem).wait()

        def wbcast(chunk, j):
            return lax.gather(chunk, jnp.full((_LANES, 1), j, jnp.int32),
                              _GDN, (1,), mode=_IN_BOUNDS)

        def pooled(rows, woff, rbase):
            def fma_rows(accs, wl, wh, r0, nrows):
                lo, hi = accs
                for j in range(nrows):
                    wvl = wbcast(wl, j)
                    wvh = wbcast(wh, j)
                    r = r0 + j
                    lo = tuple(
                        lo[c] + wvl * rows[r, pl.ds(c * _LANES, _LANES)]
                        for c in range(_DCH))
                    hi = tuple(
                        hi[c] + wvh * rows[r, pl.ds(D + c * _LANES, _LANES)]
                        for c in range(_DCH))
                return (lo, hi)

            def body(k, accs):
                wl = wlo_v[pl.ds(woff + k * _LANES, _LANES)]
                wh = whi_v[pl.ds(woff + k * _LANES, _LANES)]
                return fma_rows(accs, wl, wh, rbase + k * _LANES, _LANES)

            z = jnp.zeros((_LANES,), jnp.float32)
            accs = lax.fori_loop(0, L // _LANES, body,
                                 ((z,) * _DCH, (z,) * _DCH))
            ntail = L - _LANES * (L // _LANES)
            wl = wlo_v[pl.ds(woff + L - ntail, _LANES)]
            wh = whi_v[pl.ds(woff + L - ntail, _LANES)]
            lo, hi = fma_rows(accs, wl, wh, rbase + L - ntail, ntail)
            return tuple(lo[c] + hi[c] for c in range(_DCH))

        def compute(g, rows):
            for q in range(_PAIRS_G):
                woff = (g * _G + 2 * q) * L
                e1 = pooled(rows, woff, (2 * q) * L)
                e2 = pooled(rows, woff + L, (2 * q + 1) * L)
                for c in range(_DCH):
                    outb_v[q, pl.ds(c * _LANES, _LANES)] = (
                        jnp.abs(e1[c] - e2[c]) * (1.0 / L))
                    outb_v[q, pl.ds(D + c * _LANES, _LANES)] = (
                        (e1[c] * e2[c]) * (1.0 / (L * L)))
            pair0 = wid * (_TPW // 2) + g * _PAIRS_G
            pltpu.sync_copy(outb_v, out_hbm.at[pl.ds(pair0, _PAIRS_G)])

        issue(0, rows_a, sem_a)
        issue(1, rows_b, sem_b)

        def step(i, carry):
            g0 = 2 * i
            wait(rows_a, sem_a)
            compute(g0, rows_a)

            @pl.when(i < _NG // 2 - 1)
            def _():
                issue(g0 + 2, rows_a, sem_a)

            wait(rows_b, sem_b)
            compute(g0 + 1, rows_b)

            @pl.when(i < _NG // 2 - 1)
            def _():
                issue(g0 + 3, rows_b, sem_b)

            return carry

        lax.fori_loop(0, _NG // 2, step, 0)

    return pool


def _mlp(x, W1, b1, W2, b2, W3, b3):
    def body(x_ref, w1_ref, b1_ref, w2_ref, b2_ref, w3_ref, b3_ref, o_ref):
        f32 = jnp.float32
        w12 = jnp.dot(w1_ref[...], w2_ref[...], preferred_element_type=f32)
        w123 = jnp.dot(w12, w3_ref[...], preferred_element_type=f32)
        b12 = jnp.dot(b1_ref[...], w2_ref[...], preferred_element_type=f32) + b2_ref[...]
        beff = jnp.dot(b12, w3_ref[...], preferred_element_type=f32) + b3_ref[...]
        o_ref[...] = jnp.dot(x_ref[...], w123, preferred_element_type=f32) + beff

    return pl.pallas_call(
        body,
        out_shape=jax.ShapeDtypeStruct((B, NC_OUT), jnp.float32),
    )(x, W1, b1.reshape(1, -1), W2, b2.reshape(1, -1), W3, b3.reshape(1, -1))


def kernel(s1, s2, w1, w2, table, W1, b1, W2, b2, W3, b3):
    s_all = jnp.stack([s1.astype(jnp.int32), s2.astype(jnp.int32)],
                      axis=1).reshape(-1)
    w_all = jnp.stack([w1, w2], axis=1).reshape(-1)
    hi = s_all >= _TH
    s_half = jnp.where(hi, s_all - _TH, s_all)
    w_lo = jnp.where(hi, 0.0, w_all)
    w_hi = jnp.where(hi, w_all, 0.0)
    table2 = _repack(table.T)
    combine = _make_sc_pool()(s_half, w_lo, w_hi, table2)
    return _mlp(combine, W1, b1, W2, b2, W3, b3)
